# baseline (speedup 1.0000x reference)
import jax
import jax.numpy as jnp
from jax.experimental import pallas as pl


def _hidden_kernel(x_ref, w_ref, b_ref, out_ref):
    out_ref[:, :] = (
        jnp.dot(x_ref[:, :], w_ref[:, :].T, preferred_element_type=jnp.float32)
        + b_ref[:, :]
    )


def _spmm_kernel(adj_ref, h_ref, out_ref):
    acc = jnp.dot(adj_ref[:, :], h_ref[:, :], preferred_element_type=jnp.float32)
    out_ref[:, :] = jnp.maximum(acc, 0.0)


@jax.jit
def kernel(x, adj, W, b):
    n, d_in = x.shape
    d_out = W.shape[0]

    hidden = pl.pallas_call(
        _hidden_kernel,
        out_shape=jax.ShapeDtypeStruct((n, d_out), jnp.float32),
    )(x, W, b.reshape(1, d_out))

    bm = 400
    support = pl.pallas_call(
        _spmm_kernel,
        grid=(n // bm,),
        in_specs=[
            pl.BlockSpec((bm, n), lambda i: (i, 0)),
            pl.BlockSpec((n, d_out), lambda i: (0, 0)),
        ],
        out_specs=pl.BlockSpec((bm, d_out), lambda i: (i, 0)),
        out_shape=jax.ShapeDtypeStruct((n, d_out), jnp.float32),
    )(adj, hidden)

    return (support, adj)


# speedup vs baseline: 1.0037x; 1.0037x over previous
import jax
import jax.numpy as jnp
from jax.experimental import pallas as pl


def _hidden_kernel(x_ref, w_ref, b_ref, out_ref):
    h = (
        jnp.dot(x_ref[:, :], w_ref[:, :].T, preferred_element_type=jnp.float32)
        + b_ref[:, :]
    )
    out_ref[:, :] = h.astype(jnp.bfloat16)


def _spmm_kernel(adj_ref, h_ref, out_ref):
    acc = jnp.dot(
        adj_ref[:, :].astype(jnp.bfloat16),
        h_ref[:, :],
        preferred_element_type=jnp.float32,
    )
    out_ref[:, :] = jnp.maximum(acc, 0.0)


@jax.jit
def kernel(x, adj, W, b):
    n, d_in = x.shape
    d_out = W.shape[0]

    hidden = pl.pallas_call(
        _hidden_kernel,
        out_shape=jax.ShapeDtypeStruct((n, d_out), jnp.bfloat16),
    )(x, W, b.reshape(1, d_out))

    bm = 400
    support = pl.pallas_call(
        _spmm_kernel,
        grid=(n // bm,),
        in_specs=[
            pl.BlockSpec((bm, n), lambda i: (i, 0)),
            pl.BlockSpec((n, d_out), lambda i: (0, 0)),
        ],
        out_specs=pl.BlockSpec((bm, d_out), lambda i: (i, 0)),
        out_shape=jax.ShapeDtypeStruct((n, d_out), jnp.float32),
    )(adj, hidden)

    return (support, adj)
